# Initial kernel scaffold; baseline (speedup 1.0000x reference)
import jax
import jax.numpy as jnp
from jax.experimental import pallas as pl


def kernel(x, edge_index, edge_attr, W_e, b_e, W1, b1, W2, b2, gamma, beta):
    raise NotImplementedError("write your pallas kernel here")


# speedup vs baseline: 2.8371x; 2.8371x over previous
import functools

import jax
import jax.numpy as jnp
from jax import lax
from jax.experimental import pallas as pl
from jax.experimental.pallas import tpu as pltpu
from jax.experimental.pallas import tpu_sc as plsc

_N = 10000
_D = 128
_DE = 16
_E = 320000

_CH = 128
_NTILES = 32
_CPT = 79
_EPT = _CH * _CPT
_EPAD = _EPT * _NTILES
_NPAD = 10240
_RPS = _NPAD // 16
_BE = 4096


def _edge_proj_body(ea_ref, we_ref, be_ref, o_ref):
    o_ref[...] = lax.dot_general(
        ea_ref[...], we_ref[...], (((1,), (1,)), ((), ())),
        preferred_element_type=jnp.float32) + be_ref[...]


def _edge_proj(ea, W_e, b_e):
    return pl.pallas_call(
        _edge_proj_body,
        grid=(_EPAD // _BE,),
        in_specs=[
            pl.BlockSpec((_BE, _DE), lambda i: (i, 0)),
            pl.BlockSpec((_D, _DE), lambda i: (0, 0)),
            pl.BlockSpec((1, _D), lambda i: (0, 0)),
        ],
        out_specs=pl.BlockSpec((_BE, _D), lambda i: (i, 0)),
        out_shape=jax.ShapeDtypeStruct((_EPAD, _D), jnp.float32),
    )(ea, W_e, b_e.reshape(1, _D))


def _sc_body(x_hbm, src_hbm, dst_hbm, e_hbm, z_hbm, out_hbm,
             src_v, dst_v, x_v, e_v, agg_sh, sem_x, sem_e):
    c = lax.axis_index("c")
    s = lax.axis_index("s")
    pltpu.sync_copy(z_hbm, agg_sh.at[pl.ds(s * _RPS, _RPS)])
    plsc.subcore_barrier()

    wid = s * 2 + c
    tile_base = wid * _EPT

    def chunk(i, carry):
        base = tile_base + i * _CH
        pltpu.sync_copy(src_hbm.at[pl.ds(base, _CH)], src_v)
        pltpu.sync_copy(dst_hbm.at[pl.ds(base, _CH)], dst_v)
        cpe = pltpu.async_copy(e_hbm.at[pl.ds(base, _CH)], e_v, sem_e)
        cpx = pltpu.async_copy(x_hbm.at[src_v], x_v, sem_x)
        cpe.wait()
        cpx.wait()

        def row(r, carry2):
            for db in range(_D // 16):
                sl = pl.ds(db * 16, 16)
                x_v[r, sl] = jnp.maximum(x_v[r, sl] + e_v[r, sl], 0.0)
            return carry2

        lax.fori_loop(0, _CH, row, 0)
        pltpu.sync_copy(x_v, agg_sh.at[dst_v], add=True)
        return carry

    lax.fori_loop(0, _CPT, chunk, 0)
    plsc.subcore_barrier()
    pltpu.sync_copy(agg_sh.at[pl.ds(s * _RPS, _RPS)],
                    out_hbm.at[c, pl.ds(s * _RPS, _RPS)])


_sc_agg = pl.kernel(
    _sc_body,
    mesh=plsc.VectorSubcoreMesh(core_axis_name="c", subcore_axis_name="s"),
    out_type=jax.ShapeDtypeStruct((2, _NPAD, _D), jnp.float32),
    scratch_types=[
        pltpu.VMEM((_CH,), jnp.int32),
        pltpu.VMEM((_CH,), jnp.int32),
        pltpu.VMEM((_CH, _D), jnp.float32),
        pltpu.VMEM((_CH, _D), jnp.float32),
        pltpu.VMEM_SHARED((_NPAD, _D), jnp.float32),
        pltpu.SemaphoreType.DMA,
        pltpu.SemaphoreType.DMA,
    ],
)


def _mlp_bn_body(x_ref, p_ref, w1_ref, b1_ref, w2_ref, b2_ref, g_ref, bt_ref,
                 o_ref):
    agg = p_ref[0, :_N, :] + p_ref[1, :_N, :]
    h = x_ref[...] + agg
    h = lax.dot_general(h, w1_ref[...], (((1,), (1,)), ((), ())),
                        preferred_element_type=jnp.float32) + b1_ref[...]
    h = jnp.maximum(h, 0.0)
    h = lax.dot_general(h, w2_ref[...], (((1,), (1,)), ((), ())),
                        preferred_element_type=jnp.float32) + b2_ref[...]
    h = jnp.maximum(h, 0.0)
    mean = jnp.mean(h, axis=0, keepdims=True)
    var = jnp.mean(jnp.square(h - mean), axis=0, keepdims=True)
    o_ref[...] = (h - mean) * lax.rsqrt(var + 1e-5) * g_ref[...] + bt_ref[...]


def _mlp_bn(x, partials, W1, b1, W2, b2, gamma, beta):
    return pl.pallas_call(
        _mlp_bn_body,
        out_shape=jax.ShapeDtypeStruct((_N, _D), jnp.float32),
    )(x, partials, W1, b1.reshape(1, _D), W2, b2.reshape(1, _D),
      gamma.reshape(1, _D), beta.reshape(1, _D))


def kernel(x, edge_index, edge_attr, W_e, b_e, W1, b1, W2, b2, gamma, beta):
    src = edge_index[0]
    dst = edge_index[1]
    npad = _EPAD - _E
    fill = jnp.arange(npad, dtype=jnp.int32)
    src_p = jnp.concatenate([src, fill % _N])
    dst_p = jnp.concatenate([dst, _N + fill % (_NPAD - _N)])
    ea_p = jnp.concatenate([edge_attr, jnp.zeros((npad, _DE), jnp.float32)])

    e = _edge_proj(ea_p, W_e, b_e)
    zeros = jnp.zeros((_RPS, _D), jnp.float32)
    partials = _sc_agg(x, src_p, dst_p, e, zeros)
    return _mlp_bn(x, partials, W1, b1, W2, b2, gamma, beta)
